# baseline (speedup 1.0000x reference)
import functools

import jax
import jax.numpy as jnp
from jax import lax
from jax.experimental import pallas as pl
from jax.experimental.pallas import tpu as pltpu
from jax.experimental.pallas import tpu_sc as plsc

NUM_NODES = 100000
WINDOW = 6
DIM = 128
BATCH = 16384

NUM_CORES = 2
NUM_SUBCORES = 16
NUM_WORKERS = NUM_CORES * NUM_SUBCORES
B_PER_W = BATCH // NUM_WORKERS
CHUNK = 32
NCHUNK = B_PER_W // CHUNK
NBUF = 3


def _sc_gather_body(bank_hbm, idx_hbm, out_hbm, idx_v, *rest):
    bufs = rest[:NBUF]
    gsems = rest[NBUF:2 * NBUF]
    ssems = rest[2 * NBUF:3 * NBUF]

    wid = lax.axis_index("s") * NUM_CORES + lax.axis_index("c")
    base = wid * B_PER_W
    pltpu.sync_copy(idx_hbm.at[pl.ds(base, B_PER_W)], idx_v)

    gathers = [None] * NBUF
    scatters = [None] * NBUF

    def start_gather(c):
        b = c % NBUF
        if scatters[b] is not None:
            scatters[b].wait()
        gathers[b] = pltpu.async_copy(
            bank_hbm.at[idx_v.at[pl.ds(c * CHUNK, CHUNK)]], bufs[b], gsems[b])

    def finish_chunk(c):
        b = c % NBUF
        gathers[b].wait()
        scatters[b] = pltpu.async_copy(
            bufs[b], out_hbm.at[pl.ds(base + c * CHUNK, CHUNK)], ssems[b])

    depth = NBUF - 1
    for c in range(NCHUNK):
        start_gather(c)
        if c >= depth:
            finish_chunk(c - depth)
    for c in range(NCHUNK - depth, NCHUNK):
        finish_chunk(c)
    for b in range(NBUF):
        if scatters[b] is not None:
            scatters[b].wait()


def _sc_gather(bank, idx):
    mesh = plsc.VectorSubcoreMesh(core_axis_name="c", subcore_axis_name="s")
    k = functools.partial(
        pl.kernel,
        out_type=jax.ShapeDtypeStruct((BATCH, WINDOW, DIM), jnp.float32),
        mesh=mesh,
        compiler_params=pltpu.CompilerParams(use_tc_tiling_on_sc=True),
        scratch_types=(
            [pltpu.VMEM((B_PER_W,), jnp.int32)]
            + [pltpu.VMEM((CHUNK, WINDOW, DIM), jnp.float32)
               for _ in range(NBUF)]
            + [pltpu.SemaphoreType.DMA for _ in range(2 * NBUF)]
        ),
    )(_sc_gather_body)
    return k(bank, idx)


COPY_BLK = 1250
N_COPY_BLKS = NUM_NODES // COPY_BLK


def _bank_copy_body(in_ref, out_ref):
    out_ref[...] = in_ref[...]


def _bank_copy(bank):
    return pl.pallas_call(
        _bank_copy_body,
        grid=(N_COPY_BLKS,),
        in_specs=[pl.BlockSpec((COPY_BLK, WINDOW, DIM), lambda i: (i, 0, 0))],
        out_specs=pl.BlockSpec((COPY_BLK, WINDOW, DIM), lambda i: (i, 0, 0)),
        out_shape=jax.ShapeDtypeStruct((NUM_NODES, WINDOW, DIM), jnp.float32),
    )(bank)


ROWS_BLK = 1024


def _consistency_body(evo_ref, out_ref):
    x = evo_ref[...]
    n2 = jnp.sum(x * x, axis=-1)
    n = jnp.maximum(jnp.sqrt(n2), 1e-6)
    dot = jnp.sum(x[:, :-1, :] * x[:, 1:, :], axis=-1)
    sim = dot / (n[:, :-1] * n[:, 1:])
    mean = jnp.mean(sim, axis=-1, keepdims=True)
    var = jnp.sum((sim - mean) ** 2, axis=-1) / (WINDOW - 2)
    std = jnp.sqrt(var)
    out_ref[...] = jnp.clip(1.0 / (1.0 + std), 0.0, 1.0)[:, None]


def _consistency(evo):
    return pl.pallas_call(
        _consistency_body,
        grid=(BATCH // ROWS_BLK,),
        in_specs=[pl.BlockSpec((ROWS_BLK, WINDOW, DIM), lambda i: (i, 0, 0))],
        out_specs=pl.BlockSpec((ROWS_BLK, 1), lambda i: (i, 0)),
        out_shape=jax.ShapeDtypeStruct((BATCH, 1), jnp.float32),
    )(evo)


def kernel(bank, idx):
    bank_c = _bank_copy(bank)
    evo = _sc_gather(bank_c, idx)
    cons = _consistency(evo).reshape(BATCH)
    return evo, cons


# speedup vs baseline: 1.4822x; 1.4822x over previous
import functools

import jax
import jax.numpy as jnp
from jax import lax
from jax.experimental import pallas as pl
from jax.experimental.pallas import tpu as pltpu
from jax.experimental.pallas import tpu_sc as plsc

NUM_NODES = 100000
WINDOW = 6
DIM = 128
BATCH = 16384

NUM_CORES = 2
NUM_SUBCORES = 16
NUM_WORKERS = NUM_CORES * NUM_SUBCORES
B_PER_W = BATCH // NUM_WORKERS
CHUNK = 32
NCHUNK = B_PER_W // CHUNK
NBUF = 3


def _sc_gather_body(bank_hbm, idx_hbm, out_hbm, idx_v, *rest):
    bufs = rest[:NBUF]
    gsems = rest[NBUF:2 * NBUF]
    ssems = rest[2 * NBUF:3 * NBUF]

    wid = lax.axis_index("s") * NUM_CORES + lax.axis_index("c")
    base = wid * B_PER_W
    pltpu.sync_copy(idx_hbm.at[pl.ds(base, B_PER_W)], idx_v)

    gathers = [None] * NBUF
    scatters = [None] * NBUF

    def start_gather(c):
        b = c % NBUF
        if scatters[b] is not None:
            scatters[b].wait()
        gathers[b] = pltpu.async_copy(
            bank_hbm.at[idx_v.at[pl.ds(c * CHUNK, CHUNK)]], bufs[b], gsems[b])

    def finish_chunk(c):
        b = c % NBUF
        gathers[b].wait()
        scatters[b] = pltpu.async_copy(
            bufs[b], out_hbm.at[pl.ds(base + c * CHUNK, CHUNK)], ssems[b])

    depth = NBUF - 1
    for c in range(NCHUNK):
        start_gather(c)
        if c >= depth:
            finish_chunk(c - depth)
    for c in range(NCHUNK - depth, NCHUNK):
        finish_chunk(c)
    for b in range(NBUF):
        if scatters[b] is not None:
            scatters[b].wait()


def _sc_gather(bank, idx):
    mesh = plsc.VectorSubcoreMesh(core_axis_name="c", subcore_axis_name="s")
    k = functools.partial(
        pl.kernel,
        out_type=jax.ShapeDtypeStruct((BATCH, WINDOW, DIM), jnp.float32),
        mesh=mesh,
        compiler_params=pltpu.CompilerParams(use_tc_tiling_on_sc=True),
        scratch_types=(
            [pltpu.VMEM((B_PER_W,), jnp.int32)]
            + [pltpu.VMEM((CHUNK, WINDOW, DIM), jnp.float32)
               for _ in range(NBUF)]
            + [pltpu.SemaphoreType.DMA for _ in range(2 * NBUF)]
        ),
    )(_sc_gather_body)
    return k(bank, idx)


ROWS_BLK = 1024


def _consistency_body(evo_ref, out_ref):
    x = evo_ref[...]
    n2 = jnp.sum(x * x, axis=-1)
    n = jnp.maximum(jnp.sqrt(n2), 1e-6)
    dot = jnp.sum(x[:, :-1, :] * x[:, 1:, :], axis=-1)
    sim = dot / (n[:, :-1] * n[:, 1:])
    mean = jnp.mean(sim, axis=-1, keepdims=True)
    var = jnp.sum((sim - mean) ** 2, axis=-1) / (WINDOW - 2)
    std = jnp.sqrt(var)
    out_ref[...] = jnp.clip(1.0 / (1.0 + std), 0.0, 1.0)[:, None]


def _consistency(evo):
    return pl.pallas_call(
        _consistency_body,
        grid=(BATCH // ROWS_BLK,),
        in_specs=[pl.BlockSpec((ROWS_BLK, WINDOW, DIM), lambda i: (i, 0, 0))],
        out_specs=pl.BlockSpec((ROWS_BLK, 1), lambda i: (i, 0)),
        out_shape=jax.ShapeDtypeStruct((BATCH, 1), jnp.float32),
    )(evo)


def kernel(bank, idx):
    evo = _sc_gather(bank, idx)
    cons = _consistency(evo).reshape(BATCH)
    return evo, cons
